# baseline (device time: 43406 ns/iter reference)
import jax
import jax.numpy as jnp
from jax import lax
from jax.experimental import pallas as pl
from jax.experimental.pallas import tpu as pltpu

N_DEV = 4
B, Sq, Skv, Hq, Dh = 2, 512, 512, 32, 64
H_LOC = Hq // N_DEV
D_LOC = H_LOC * Dh
D_MODEL = 768
CHUNK = (B * Sq) // N_DEV
HALF = CHUNK // 2
BLK = 64
FROM_LEFT, FROM_RIGHT, FROM_DIAG = 0, 1, 2


def kernel(x, Wq, K_ext, V_ext, Wo):
    x16 = x.astype(jnp.bfloat16)
    wq16 = (Wq * 0.125).astype(jnp.bfloat16)
    k16 = jnp.transpose(K_ext, (0, 2, 1, 3)).astype(jnp.bfloat16)
    v16 = jnp.transpose(V_ext, (0, 2, 1, 3)).astype(jnp.bfloat16)
    wo16 = Wo.astype(jnp.bfloat16)

    def body(x_ref, wq_ref, k_ref, v_ref, wo_ref, out_ref,
             part_s, rs_comm, ag_comm,
             rs_send, rs_recv, ag_send, ag_recv):
        my_i = lax.axis_index("i")
        left = lax.rem(my_i + N_DEV - 1, N_DEV)
        right = lax.rem(my_i + 1, N_DEV)
        diag = lax.rem(my_i + 2, N_DEV)

        barrier_sem = pltpu.get_barrier_semaphore()
        for nbr in (left, right, diag):
            pl.semaphore_signal(barrier_sem, inc=1, device_id=(nbr,),
                                device_id_type=pl.DeviceIdType.MESH)
        pl.semaphore_wait(barrier_sem, 3)

        wq_loc = wq_ref[:, pl.ds(my_i * D_LOC, D_LOC)]
        wo_loc = wo_ref[pl.ds(my_i * D_LOC, D_LOC), :]

        def attn_bands(c, b, q16, s0g):
            band_ctx = []
            for t in range(2):
                r0 = s0g + t * HALF
                kvl = r0 + HALF
                q_t = q16[t * HALF:(t + 1) * HALF, :]
                row_blk = (lax.broadcasted_iota(jnp.int32, (HALF, kvl), 0)
                           + r0) // BLK
                col_blk = lax.broadcasted_iota(
                    jnp.int32, (HALF, kvl), 1) // BLK
                mask = col_blk <= row_blk
                ctx_cols = []
                for h in range(H_LOC):
                    q_h = q_t[:, h * Dh:(h + 1) * Dh]
                    k_h = k_ref[b, h, :kvl, :]
                    v_h = v_ref[b, h, :kvl, :]
                    s = lax.dot_general(
                        q_h, k_h, (((1,), (1,)), ((), ())),
                        preferred_element_type=jnp.float32)
                    e = jnp.exp(jnp.where(mask, s, -1e9))
                    denom = jnp.sum(e, axis=-1, keepdims=True)
                    ctx_h = lax.dot_general(
                        e.astype(jnp.bfloat16), v_h,
                        (((1,), (0,)), ((), ())),
                        preferred_element_type=jnp.float32)
                    ctx_cols.append((ctx_h / denom).astype(jnp.bfloat16))
                band_ctx.append(jnp.concatenate(ctx_cols, axis=-1))
            ctx = jnp.concatenate(band_ctx, axis=0)
            part = jnp.dot(ctx, wo_loc,
                           preferred_element_type=jnp.float32)
            part_s[c] = part.astype(jnp.bfloat16)

        def compute_chunk(c):
            b = lax.div(c, 2)
            par = lax.rem(c, 2)
            xb = x_ref[b, pl.ds(par * CHUNK, CHUNK), :]
            q = jnp.dot(xb, wq_loc,
                        preferred_element_type=jnp.float32)
            q16 = q.astype(jnp.bfloat16)

            @pl.when(par == 0)
            def _():
                attn_bands(c, b, q16, 0)

            @pl.when(par == 1)
            def _():
                attn_bands(c, b, q16, CHUNK)

        def out_store(c, val16):
            b = lax.div(c, 2)
            s0 = lax.rem(c, 2) * CHUNK
            out_ref[b, pl.ds(s0, CHUNK), :] = val16.astype(jnp.float32)

        def send(src, comm, slot, sends, recvs, target):
            rdma = pltpu.make_async_remote_copy(
                src_ref=src, dst_ref=comm.at[slot],
                send_sem=sends.at[slot], recv_sem=recvs.at[slot],
                device_id=(target,), device_id_type=pl.DeviceIdType.MESH)
            rdma.start()
            return rdma

        compute_chunk(right)
        s_r = send(part_s.at[right], rs_comm, FROM_LEFT,
                   rs_send, rs_recv, right)
        compute_chunk(left)
        s_l = send(part_s.at[left], rs_comm, FROM_RIGHT,
                   rs_send, rs_recv, left)
        compute_chunk(diag)
        s_d = send(part_s.at[diag], rs_comm, FROM_DIAG,
                   rs_send, rs_recv, diag)
        compute_chunk(my_i)

        s_r.wait_recv()
        s_l.wait_recv()
        s_d.wait_recv()
        own = (part_s[my_i].astype(jnp.float32)
               + rs_comm[FROM_LEFT].astype(jnp.float32)
               + rs_comm[FROM_RIGHT].astype(jnp.float32)
               + rs_comm[FROM_DIAG].astype(jnp.float32))
        own16 = own.astype(jnp.bfloat16)
        part_s[my_i] = own16

        a_r = send(part_s.at[my_i], ag_comm, FROM_LEFT,
                   ag_send, ag_recv, right)
        a_l = send(part_s.at[my_i], ag_comm, FROM_RIGHT,
                   ag_send, ag_recv, left)
        a_d = send(part_s.at[my_i], ag_comm, FROM_DIAG,
                   ag_send, ag_recv, diag)
        out_store(my_i, own16)

        a_r.wait_recv()
        out_store(left, ag_comm[FROM_LEFT])
        a_l.wait_recv()
        out_store(right, ag_comm[FROM_RIGHT])
        a_d.wait_recv()
        out_store(diag, ag_comm[FROM_DIAG])

        for rdma in (s_r, s_l, s_d, a_r, a_l, a_d):
            rdma.wait_send()

    return pl.pallas_call(
        body,
        out_shape=jax.ShapeDtypeStruct((B, Sq, D_MODEL), jnp.float32),
        in_specs=[pl.BlockSpec(memory_space=pltpu.VMEM)] * 5,
        out_specs=pl.BlockSpec(memory_space=pltpu.VMEM),
        scratch_shapes=[
            pltpu.VMEM((N_DEV, CHUNK, D_MODEL), jnp.bfloat16),
            pltpu.VMEM((3, CHUNK, D_MODEL), jnp.bfloat16),
            pltpu.VMEM((3, CHUNK, D_MODEL), jnp.bfloat16),
            pltpu.SemaphoreType.DMA((3,)),
            pltpu.SemaphoreType.DMA((3,)),
            pltpu.SemaphoreType.DMA((3,)),
            pltpu.SemaphoreType.DMA((3,)),
        ],
        compiler_params=pltpu.CompilerParams(collective_id=0),
    )(x16, wq16, k16, v16, wo16)
